# baseline (speedup 1.0000x reference)
import functools

import jax
import jax.numpy as jnp
from jax import lax
from jax.experimental import pallas as pl
from jax.experimental.pallas import tpu as pltpu
from jax.experimental.pallas import tpu_sc as plsc

NC, NS = 2, 16
NW = NC * NS
G = 128
NBUF = 2
PT = 64


def _pair_table_kernel(x_ref, g_ref, b_ref, o_ref):
    x = x_ref[...]
    mean = jnp.mean(x, axis=1, keepdims=True)
    c = x - mean
    var = jnp.mean(c * c, axis=1, keepdims=True)
    normed = c * lax.rsqrt(var + 1e-12) * g_ref[...] + b_ref[...]
    d = x.shape[1]
    rows = lax.broadcasted_iota(jnp.int32, (PT, 1), 0)
    for half, ridx in ((0, rows >> 3), (1, rows & 7)):
        acc = jnp.broadcast_to(normed[0:1, :], (PT, d))
        for v_ in range(1, 8):
            acc = jnp.where(ridx == v_, jnp.broadcast_to(
                normed[v_:v_ + 1, :], (PT, d)), acc)
        o_ref[:, half * d:(half + 1) * d] = acc


def _make_sc_expand(n_pairs: int, d2: int):
    assert n_pairs % (NW * G) == 0
    per_w = n_pairs // NW
    chunks = per_w // G
    assert chunks % NBUF == 0

    mesh = plsc.VectorSubcoreMesh(core_axis_name="c", subcore_axis_name="s")

    @functools.partial(
        pl.kernel,
        mesh=mesh,
compiler_params=pltpu.CompilerParams(needs_layout_passes=False),
        out_type=jax.ShapeDtypeStruct((n_pairs, 2, d2 // 2), jnp.float32),
        scratch_types=[
            pltpu.VMEM((chunks, G), jnp.int32),
            pltpu.VMEM_SHARED((PT, 2, d2 // 2), jnp.float32),
            pltpu.VMEM((G, 2, d2 // 2), jnp.float32),
            pltpu.VMEM((G, 2, d2 // 2), jnp.float32),
            pltpu.SemaphoreType.DMA,
            pltpu.SemaphoreType.DMA,
            pltpu.SemaphoreType.DMA,
        ],
    )
    def sc_expand(table_hbm, idx_hbm, out_hbm, idx_v, table_sh, ob0, ob1,
                  w0, w1, gsem):
        outbufs = (ob0, ob1)
        wsems = (w0, w1)
        cid = lax.axis_index("c")
        sid = lax.axis_index("s")
        wid = sid * NC + cid
        base = wid * per_w

        @pl.when(sid == 0)
        def _():
            pltpu.sync_copy(table_hbm, table_sh)

        pltpu.sync_copy(idx_hbm.at[wid], idx_v)
        plsc.subcore_barrier()

        def fill_chunk(chunk, b):
            pltpu.async_copy(
                table_sh.at[idx_v.at[chunk]], outbufs[b], gsem).wait()

        def start_write(chunk, b):
            pltpu.async_copy(
                outbufs[b], out_hbm.at[pl.ds(base + chunk * G, G)], wsems[b])

        def wait_write(b):
            pltpu.make_async_copy(
                outbufs[b], out_hbm.at[pl.ds(0, G)], wsems[b]).wait()

        for b in range(NBUF):
            fill_chunk(b, b)
            start_write(b, b)

        def outer(go, carry):
            for b in range(NBUF):
                chunk = go * NBUF + b
                wait_write(b)
                fill_chunk(chunk, b)
                start_write(chunk, b)
            return carry

        lax.fori_loop(1, chunks // NBUF, outer, 0)
        for b in range(NBUF):
            wait_write(b)

    return sc_expand


def kernel(snp, is_padding, emb_table, ln_gamma, ln_beta):
    b, l = snp.shape
    v, d = emb_table.shape
    n = b * l

    table8 = jnp.zeros((8, d), jnp.float32).at[:v].set(emb_table)
    pair_table = pl.pallas_call(
        _pair_table_kernel,
        out_shape=jax.ShapeDtypeStruct((PT, 2 * d), jnp.float32),
    )(table8, ln_gamma.reshape(1, d), ln_beta.reshape(1, d))

    sp = snp.astype(jnp.int32).reshape(n // 2, 2)
    pidx = (sp[:, 0] << 3) | sp[:, 1]
    idx = pidx.reshape(NW, n // (2 * NW * G), G)
    out = _make_sc_expand(n // 2, 2 * d)(pair_table.reshape(PT, 2, d), idx)
    return out.reshape(b, l, d), is_padding


# speedup vs baseline: 1.8620x; 1.8620x over previous
import functools

import jax
import jax.numpy as jnp
from jax import lax
from jax.experimental import pallas as pl
from jax.experimental.pallas import tpu as pltpu
from jax.experimental.pallas import tpu_sc as plsc

NC, NS = 2, 16
NW = NC * NS
G = 128
GPC = 2
C = G * GPC
NBUF = 2


def _ln_table_kernel(x_ref, g_ref, b_ref, o_ref):
    x = x_ref[...]
    mean = jnp.mean(x, axis=1, keepdims=True)
    c = x - mean
    var = jnp.mean(c * c, axis=1, keepdims=True)
    o_ref[...] = c * lax.rsqrt(var + 1e-12) * g_ref[...] + b_ref[...]


def _make_sc_expand(n_tokens: int, d: int):
    assert n_tokens % (NW * C) == 0
    per_w = n_tokens // NW
    chunks = per_w // C
    groups = per_w // G
    assert chunks % NBUF == 0

    mesh = plsc.VectorSubcoreMesh(core_axis_name="c", subcore_axis_name="s")

    @functools.partial(
        pl.kernel,
        mesh=mesh,
        compiler_params=pltpu.CompilerParams(needs_layout_passes=False),
        out_type=jax.ShapeDtypeStruct((n_tokens, d), jnp.float32),
        scratch_types=[
            pltpu.VMEM((groups, G), jnp.int32),
            pltpu.VMEM_SHARED((8, d), jnp.float32),
            pltpu.VMEM((C, d), jnp.float32),
            pltpu.VMEM((C, d), jnp.float32),
            pltpu.SemaphoreType.DMA,
            pltpu.SemaphoreType.DMA,
            pltpu.SemaphoreType.DMA,
        ],
    )
    def sc_expand(table_hbm, idx_hbm, out_hbm,
                  idx_v, table_sh, ob0, ob1, w0, w1, gsem):
        outbufs = (ob0, ob1)
        wsems = (w0, w1)
        cid = lax.axis_index("c")
        sid = lax.axis_index("s")
        wid = sid * NC + cid
        base = wid * per_w

        @pl.when(sid == 0)
        def _():
            pltpu.sync_copy(table_hbm, table_sh)

        pltpu.sync_copy(idx_hbm.at[wid], idx_v)
        plsc.subcore_barrier()

        def fill_chunk(chunk, b):
            ob = outbufs[b]
            g0 = chunk * GPC
            cps = [
                pltpu.async_copy(
                    table_sh.at[idx_v.at[g0 + i]],
                    ob.at[pl.ds(i * G, G)], gsem)
                for i in range(GPC)
            ]
            for cp in cps:
                cp.wait()

        def start_write(chunk, b):
            pltpu.async_copy(
                outbufs[b], out_hbm.at[pl.ds(base + chunk * C, C)], wsems[b])

        def wait_write(b):
            pltpu.make_async_copy(
                outbufs[b], out_hbm.at[pl.ds(0, C)], wsems[b]).wait()

        for b in range(NBUF):
            fill_chunk(b, b)
            start_write(b, b)

        def outer(go, carry):
            for b in range(NBUF):
                chunk = go * NBUF + b
                wait_write(b)
                fill_chunk(chunk, b)
                start_write(chunk, b)
            return carry

        lax.fori_loop(1, chunks // NBUF, outer, 0)
        for b in range(NBUF):
            wait_write(b)

    return sc_expand


def kernel(snp, is_padding, emb_table, ln_gamma, ln_beta):
    b, l = snp.shape
    v, d = emb_table.shape
    n = b * l

    table8 = jnp.zeros((8, d), jnp.float32).at[:v].set(emb_table)
    normed = pl.pallas_call(
        _ln_table_kernel,
        out_shape=jax.ShapeDtypeStruct((8, d), jnp.float32),
    )(table8, ln_gamma.reshape(1, d), ln_beta.reshape(1, d))

    idx = snp.astype(jnp.int32).reshape(NW, n // (NW * G), G)
    out = _make_sc_expand(n, d)(normed, idx)
    return out.reshape(b, l, d), is_padding
